# baseline (speedup 1.0000x reference)
import functools

import jax
import jax.numpy as jnp
from jax import lax
from jax.experimental import pallas as pl
from jax.experimental.pallas import tpu as pltpu
from jax.experimental.pallas import tpu_sc as plsc

_B = 1024
_NPG = 54
_NP = 64
_NR = 128
_EPG = 864
_D_IN = 64
_D_H = 128
_D_O = 4
_GLOB = 32

_G_CONV = 32


def _build_adj(edge_index, w, g0, gn):
    info = plsc.get_sparse_core_info()
    n_workers = info.num_cores * info.num_subcores
    gpw = gn // n_workers
    epw = gpw * _EPG
    mesh = plsc.VectorSubcoreMesh(core_axis_name="c", subcore_axis_name="s")

    @functools.partial(
        pl.kernel,
        out_type=jax.ShapeDtypeStruct((gn, _NP * _NR), jnp.float32),
        mesh=mesh,
        scratch_types=[
            pltpu.VMEM((epw,), jnp.int32),
            pltpu.VMEM((epw,), jnp.int32),
            pltpu.VMEM((epw,), jnp.float32),
            pltpu.VMEM((_NP * _NR,), jnp.float32),
            pltpu.VMEM((_NP * _NR,), jnp.float32),
            pltpu.SemaphoreType.DMA,
            pltpu.SemaphoreType.DMA,
            pltpu.SemaphoreType.DMA,
            pltpu.SemaphoreType.DMA,
        ],
        compiler_params=pltpu.CompilerParams(
            needs_layout_passes=False, skip_device_barrier=True),
        cost_estimate=pl.CostEstimate(
            flops=2_000_000, bytes_accessed=45_000_000, transcendentals=0),
    )
    def build(ei_hbm, w_hbm, a_hbm, src_v, dst_v, w_v, acc0, acc1,
              sem_s, sem_d, sem_w, sem_o):
        wid = lax.axis_index("s") * info.num_cores + lax.axis_index("c")
        ebase = g0 * _EPG + wid * epw
        cs = pltpu.async_copy(ei_hbm.at[0, pl.ds(ebase, epw)], src_v, sem_s)
        cd = pltpu.async_copy(ei_hbm.at[1, pl.ds(ebase, epw)], dst_v, sem_d)
        cw = pltpu.async_copy(w_hbm.at[pl.ds(ebase, epw)], w_v, sem_w)

        def zero_all(j, c):
            for u in range(4):
                acc0[pl.ds(j * 64 + u * 16, 16)] = jnp.zeros(
                    (16,), jnp.float32)
                acc1[pl.ds(j * 64 + u * 16, 16)] = jnp.zeros(
                    (16,), jnp.float32)
            return c

        lax.fori_loop(0, _NP * _NR // 64, zero_all, 0)
        cs.wait()
        cd.wait()
        cw.wait()

        accs = (acc0, acc1)
        pending = [None, None]
        for gi in range(gpw):
            acc = accs[gi % 2]
            if pending[gi % 2] is not None:
                pending[gi % 2].wait()

            if gi >= 2:
                def zero_rows(d, c, acc=acc):
                    for u in range(4):
                        acc[pl.ds(d * _NR + u * 16, 16)] = jnp.zeros(
                            (16,), jnp.float32)
                    return c

                lax.fori_loop(0, _NPG, zero_rows, 0)

            g = wid * gpw + gi
            goff = (g0 + g) * _NPG
            e0 = gi * _EPG

            def edge48(i, c, acc=acc, e0=e0, goff=goff):
                for u in range(3):
                    o = e0 + i * 48 + u * 16
                    s = src_v[pl.ds(o, 16)]
                    d = dst_v[pl.ds(o, 16)]
                    ww = w_v[pl.ds(o, 16)]
                    idx = (d - goff) * _NR + (s - goff)
                    plsc.addupdate_scatter(acc, [idx], ww)
                return c

            lax.fori_loop(0, _EPG // 48, edge48, 0)
            pending[gi % 2] = pltpu.async_copy(acc, a_hbm.at[g], sem_o)
        for p in pending:
            if p is not None:
                p.wait()

    return build(edge_index, w)


def _net_body(a_ref, x_ref, gf_ref, w1cat_ref, br1_ref, w2cat_ref,
              br2_ref, wg1_ref, bg1_ref, wg2_ref, bg2_ref, wg3_ref,
              bg3_ref, w2e_ref, w1g_ref, bo1_ref, wo2_ref, bo2_ref, out_ref,
              cat1_s, cat2_s):
    pad_src = jnp.zeros((_NR - _NPG, _D_IN), jnp.float32)
    for r in range(_G_CONV):
        x_r = x_ref[pl.ds(r * _NPG, _NPG), :]
        xp128 = jnp.concatenate([x_r, pad_src], axis=0)
        a_r = jnp.reshape(a_ref[r], (_NP, _NR))
        cat1_s[pl.ds(r * _NP, _NP), :_D_IN] = jnp.dot(
            a_r, xp128, preferred_element_type=jnp.float32)
        cat1_s[pl.ds(r * _NP, _NP), _D_IN:] = xp128[:_NP]
    h1 = jnp.maximum(
        jnp.dot(cat1_s[...], w1cat_ref[...],
                preferred_element_type=jnp.float32) + br1_ref[...],
        0.0)
    cat2_s[:, _D_H:] = h1
    pad_h = jnp.zeros((_NR - _NP, _D_H), jnp.float32)
    for r in range(_G_CONV):
        a_r = jnp.reshape(a_ref[r], (_NP, _NR))
        h1p = jnp.concatenate(
            [cat2_s[pl.ds(r * _NP, _NP), _D_H:], pad_h], axis=0)
        cat2_s[pl.ds(r * _NP, _NP), :_D_H] = jnp.dot(
            a_r, h1p, preferred_element_type=jnp.float32)
    h2 = jnp.maximum(
        jnp.dot(cat2_s[...], w2cat_ref[...],
                preferred_element_type=jnp.float32) + br2_ref[...],
        0.0)
    row_ok = (lax.broadcasted_iota(jnp.int32, (_G_CONV * _NP, _D_O), 0)
              % _NP) < _NPG
    h2 = jnp.where(row_ok, h2, 0.0)

    h2t = jnp.transpose(h2, (1, 0))
    m = jnp.reshape(h2t, (_D_O, _G_CONV // 2, 2 * _NP))
    z2 = jnp.zeros((_G_CONV // 2, 2 * _D_H), jnp.float32)
    for c in range(_D_O):
        z2 = z2 + jnp.dot(m[c], w2e_ref[c],
                          preferred_element_type=jnp.float32)
    ze = jnp.reshape(z2, (_G_CONV, _D_H))

    gv = gf_ref[...]
    gv = jnp.maximum(jnp.dot(gv, wg1_ref[...],
                             preferred_element_type=jnp.float32)
                     + bg1_ref[...], 0.0)
    gv = jnp.maximum(jnp.dot(gv, wg2_ref[...],
                             preferred_element_type=jnp.float32)
                     + bg2_ref[...], 0.0)
    gv = jnp.maximum(jnp.dot(gv, wg3_ref[...],
                             preferred_element_type=jnp.float32)
                     + bg3_ref[...], 0.0)

    z = jnp.maximum(
        ze + jnp.dot(gv, w1g_ref[...], preferred_element_type=jnp.float32)
        + bo1_ref[...], 0.0)
    z = jnp.dot(z, wo2_ref[...], preferred_element_type=jnp.float32) \
        + bo2_ref[...]
    out_ref[...] = jax.nn.sigmoid(z)


def _net(a, x, gf, w_rel1, b_rel1, w_root1, w_rel2, b_rel2, w_root2,
         wg1, bg1, wg2, bg2, wg3, bg3, w2e, w1g, bo1, wo2, bo2,
         interpret=False):
    gn = a.shape[0]
    grid = (gn // _G_CONV,)
    row = lambda shape: pl.BlockSpec(shape, lambda i: (i, 0))
    full2 = lambda shape: pl.BlockSpec(shape, lambda i: (0, 0))
    w1cat = jnp.concatenate([w_rel1, w_root1], axis=0)
    w2cat = jnp.concatenate([w_rel2, w_root2], axis=0)
    return pl.pallas_call(
        _net_body,
        grid=grid,
        in_specs=[
            row((_G_CONV, _NP * _NR)),
            row((_G_CONV * _NPG, _D_IN)),
            row((_G_CONV, _GLOB)),
            full2((2 * _D_IN, _D_H)),
            full2((1, _D_H)),
            full2((2 * _D_H, _D_O)),
            full2((1, _D_O)),
            full2((_GLOB, 8)),
            full2((1, 8)),
            full2((8, 8)),
            full2((1, 8)),
            full2((8, _GLOB)),
            full2((1, _GLOB)),
            pl.BlockSpec((_D_O, _NR, 2 * _D_H), lambda i: (0, 0, 0)),
            full2((_GLOB, _D_H)),
            full2((1, _D_H)),
            full2((_D_H, 1)),
            full2((1, 1)),
        ],
        out_specs=row((_G_CONV, 1)),
        out_shape=jax.ShapeDtypeStruct((gn, 1), jnp.float32),
        scratch_shapes=[
            pltpu.VMEM((_G_CONV * _NP, 2 * _D_IN), jnp.float32),
            pltpu.VMEM((_G_CONV * _NP, 2 * _D_H), jnp.float32),
        ],
        interpret=interpret,
    )(a, x, gf, w1cat, b_rel1.reshape(1, _D_H),
      w2cat, b_rel2.reshape(1, _D_O),
      wg1, bg1.reshape(1, 8), wg2, bg2.reshape(1, 8), wg3,
      bg3.reshape(1, _GLOB), w2e, w1g, bo1.reshape(1, _D_H), wo2,
      bo2.reshape(1, 1))


def _prep_head_weights(Wo1):
    w1r = Wo1[:_NPG * _D_O].reshape(_NPG, _D_O, _D_H)
    base = jnp.pad(w1r, ((0, _NP - _NPG), (0, 0), (0, 0)))
    base = base.transpose(1, 0, 2)
    zblk = jnp.zeros((_D_O, _NP, _D_H), jnp.float32)
    top = jnp.concatenate([base, zblk], axis=2)
    bot = jnp.concatenate([zblk, base], axis=2)
    return jnp.concatenate([top, bot], axis=1)


def kernel(x, edge_index, edge_attr, globalFeats, isTrain, W_rel1, b_rel1,
           W_root1, W_rel2, b_rel2, W_root2, Wg1, bg1, Wg2, bg2, Wg3, bg3,
           Wo1, bo1, Wo2, bo2):
    w2e = _prep_head_weights(Wo1)
    w1g = Wo1[_NPG * _D_O:]

    half = _B // 2
    outs = []
    for h in range(2):
        a_h = _build_adj(edge_index, edge_attr, h * half, half)
        x_h = lax.slice(x, (h * half * _NPG, 0), ((h + 1) * half * _NPG,
                                                  _D_IN))
        gf_h = lax.slice(globalFeats, (h * half, 0), ((h + 1) * half,
                                                      _GLOB))
        outs.append(_net(a_h, x_h, gf_h, W_rel1, b_rel1, W_root1,
                         W_rel2, b_rel2, W_root2, Wg1, bg1, Wg2, bg2,
                         Wg3, bg3, w2e, w1g, bo1, Wo2, bo2))
    return jnp.concatenate(outs, axis=0)


# speedup vs baseline: 1.1348x; 1.1348x over previous
import functools

import jax
import jax.numpy as jnp
from jax import lax
from jax.experimental import pallas as pl
from jax.experimental.pallas import tpu as pltpu
from jax.experimental.pallas import tpu_sc as plsc

_B = 1024
_NPG = 54
_NP = 64
_NR = 128
_EPG = 864
_D_IN = 64
_D_H = 128
_D_O = 4
_GLOB = 32

_G_CONV = 32


def _build_adj(edge_index, w, g0, gn):
    info = plsc.get_sparse_core_info()
    n_workers = info.num_cores * info.num_subcores
    gpw = gn // n_workers
    epw = gpw * _EPG
    mesh = plsc.VectorSubcoreMesh(core_axis_name="c", subcore_axis_name="s")

    @functools.partial(
        pl.kernel,
        out_type=jax.ShapeDtypeStruct((gn, _NP * _NR), jnp.float32),
        mesh=mesh,
        scratch_types=[
            pltpu.VMEM((epw,), jnp.int32),
            pltpu.VMEM((epw,), jnp.int32),
            pltpu.VMEM((epw,), jnp.float32),
            pltpu.VMEM((_NP * _NR,), jnp.float32),
            pltpu.VMEM((_NP * _NR,), jnp.float32),
            pltpu.SemaphoreType.DMA,
            pltpu.SemaphoreType.DMA,
            pltpu.SemaphoreType.DMA,
            pltpu.SemaphoreType.DMA,
        ],
        compiler_params=pltpu.CompilerParams(
            needs_layout_passes=False, skip_device_barrier=True),
        cost_estimate=pl.CostEstimate(
            flops=2_000_000, bytes_accessed=45_000_000, transcendentals=0),
    )
    def build(ei_hbm, w_hbm, a_hbm, src_v, dst_v, w_v, acc0, acc1,
              sem_s, sem_d, sem_w, sem_o):
        wid = lax.axis_index("s") * info.num_cores + lax.axis_index("c")
        ebase = g0 * _EPG + wid * epw
        cs = pltpu.async_copy(ei_hbm.at[0, pl.ds(ebase, epw)], src_v, sem_s)
        cd = pltpu.async_copy(ei_hbm.at[1, pl.ds(ebase, epw)], dst_v, sem_d)
        cw = pltpu.async_copy(w_hbm.at[pl.ds(ebase, epw)], w_v, sem_w)

        def zero_all(j, c):
            for u in range(4):
                acc0[pl.ds(j * 64 + u * 16, 16)] = jnp.zeros(
                    (16,), jnp.float32)
                acc1[pl.ds(j * 64 + u * 16, 16)] = jnp.zeros(
                    (16,), jnp.float32)
            return c

        lax.fori_loop(0, _NP * _NR // 64, zero_all, 0)
        cs.wait()
        cd.wait()
        cw.wait()

        accs = (acc0, acc1)
        pending = [None, None]
        for gi in range(gpw):
            acc = accs[gi % 2]
            if pending[gi % 2] is not None:
                pending[gi % 2].wait()

            if gi >= 2:
                def zero_rows(d, c, acc=acc):
                    for u in range(4):
                        acc[pl.ds(d * _NR + u * 16, 16)] = jnp.zeros(
                            (16,), jnp.float32)
                    return c

                lax.fori_loop(0, _NPG, zero_rows, 0)

            g = wid * gpw + gi
            goff = (g0 + g) * _NPG
            e0 = gi * _EPG

            def edge48(i, c, acc=acc, e0=e0, goff=goff):
                for u in range(3):
                    o = e0 + i * 48 + u * 16
                    s = src_v[pl.ds(o, 16)]
                    d = dst_v[pl.ds(o, 16)]
                    ww = w_v[pl.ds(o, 16)]
                    idx = (d - goff) * _NR + (s - goff)
                    plsc.addupdate_scatter(acc, [idx], ww)
                return c

            lax.fori_loop(0, _EPG // 48, edge48, 0)
            pending[gi % 2] = pltpu.async_copy(acc, a_hbm.at[g], sem_o)
        for p in pending:
            if p is not None:
                p.wait()

    return build(edge_index, w)


def _net_body(a_ref, x_ref, gf_ref, w1cat_ref, br1_ref, w2cat_ref,
              br2_ref, wg1_ref, bg1_ref, wg2_ref, bg2_ref, wg3_ref,
              bg3_ref, w2e_ref, w1g_ref, bo1_ref, wo2_ref, bo2_ref, out_ref,
              cat1_s, cat2_s):
    pad_src = jnp.zeros((_NR - _NPG, _D_IN), jnp.float32)
    for r in range(_G_CONV):
        x_r = x_ref[pl.ds(r * _NPG, _NPG), :]
        xp128 = jnp.concatenate([x_r, pad_src], axis=0)
        a_r = jnp.reshape(a_ref[r], (_NP, _NR))
        cat1_s[pl.ds(r * _NP, _NP), :_D_IN] = jnp.dot(
            a_r, xp128, preferred_element_type=jnp.float32)
        cat1_s[pl.ds(r * _NP, _NP), _D_IN:] = xp128[:_NP]
    h1 = jnp.maximum(
        jnp.dot(cat1_s[...], w1cat_ref[...],
                preferred_element_type=jnp.float32) + br1_ref[...],
        0.0)
    cat2_s[:, _D_H:] = h1
    pad_h = jnp.zeros((_NR - _NP, _D_H), jnp.float32)
    for r in range(_G_CONV):
        a_r = jnp.reshape(a_ref[r], (_NP, _NR))
        h1p = jnp.concatenate(
            [cat2_s[pl.ds(r * _NP, _NP), _D_H:], pad_h], axis=0)
        cat2_s[pl.ds(r * _NP, _NP), :_D_H] = jnp.dot(
            a_r, h1p, preferred_element_type=jnp.float32)
    h2 = jnp.maximum(
        jnp.dot(cat2_s[...], w2cat_ref[...],
                preferred_element_type=jnp.float32) + br2_ref[...],
        0.0)
    row_ok = (lax.broadcasted_iota(jnp.int32, (_G_CONV * _NP, _D_O), 0)
              % _NP) < _NPG
    h2 = jnp.where(row_ok, h2, 0.0)

    h2t = jnp.transpose(h2, (1, 0))
    m = jnp.reshape(h2t, (_D_O, _G_CONV // 2, 2 * _NP))
    z2 = jnp.zeros((_G_CONV // 2, 2 * _D_H), jnp.float32)
    for c in range(_D_O):
        z2 = z2 + jnp.dot(m[c], w2e_ref[c],
                          preferred_element_type=jnp.float32)
    ze = jnp.reshape(z2, (_G_CONV, _D_H))

    gv = gf_ref[...]
    gv = jnp.maximum(jnp.dot(gv, wg1_ref[...],
                             preferred_element_type=jnp.float32)
                     + bg1_ref[...], 0.0)
    gv = jnp.maximum(jnp.dot(gv, wg2_ref[...],
                             preferred_element_type=jnp.float32)
                     + bg2_ref[...], 0.0)
    gv = jnp.maximum(jnp.dot(gv, wg3_ref[...],
                             preferred_element_type=jnp.float32)
                     + bg3_ref[...], 0.0)

    z = jnp.maximum(
        ze + jnp.dot(gv, w1g_ref[...], preferred_element_type=jnp.float32)
        + bo1_ref[...], 0.0)
    z = jnp.dot(z, wo2_ref[...], preferred_element_type=jnp.float32) \
        + bo2_ref[...]
    out_ref[...] = jax.nn.sigmoid(z)


def _net(a, x, gf, w_rel1, b_rel1, w_root1, w_rel2, b_rel2, w_root2,
         wg1, bg1, wg2, bg2, wg3, bg3, w2e, w1g, bo1, wo2, bo2,
         interpret=False):
    gn = a.shape[0]
    grid = (gn // _G_CONV,)
    row = lambda shape: pl.BlockSpec(shape, lambda i: (i, 0))
    full2 = lambda shape: pl.BlockSpec(shape, lambda i: (0, 0))
    w1cat = jnp.concatenate([w_rel1, w_root1], axis=0)
    w2cat = jnp.concatenate([w_rel2, w_root2], axis=0)
    return pl.pallas_call(
        _net_body,
        grid=grid,
        in_specs=[
            row((_G_CONV, _NP * _NR)),
            row((_G_CONV * _NPG, _D_IN)),
            row((_G_CONV, _GLOB)),
            full2((2 * _D_IN, _D_H)),
            full2((1, _D_H)),
            full2((2 * _D_H, _D_O)),
            full2((1, _D_O)),
            full2((_GLOB, 8)),
            full2((1, 8)),
            full2((8, 8)),
            full2((1, 8)),
            full2((8, _GLOB)),
            full2((1, _GLOB)),
            pl.BlockSpec((_D_O, _NR, 2 * _D_H), lambda i: (0, 0, 0)),
            full2((_GLOB, _D_H)),
            full2((1, _D_H)),
            full2((_D_H, 1)),
            full2((1, 1)),
        ],
        out_specs=row((_G_CONV, 1)),
        out_shape=jax.ShapeDtypeStruct((gn, 1), jnp.float32),
        scratch_shapes=[
            pltpu.VMEM((_G_CONV * _NP, 2 * _D_IN), jnp.float32),
            pltpu.VMEM((_G_CONV * _NP, 2 * _D_H), jnp.float32),
        ],
        interpret=interpret,
    )(a, x, gf, w1cat, b_rel1.reshape(1, _D_H),
      w2cat, b_rel2.reshape(1, _D_O),
      wg1, bg1.reshape(1, 8), wg2, bg2.reshape(1, 8), wg3,
      bg3.reshape(1, _GLOB), w2e, w1g, bo1.reshape(1, _D_H), wo2,
      bo2.reshape(1, 1))


def _prep_head_weights(Wo1):
    w1r = Wo1[:_NPG * _D_O].reshape(_NPG, _D_O, _D_H)
    base = jnp.pad(w1r, ((0, _NP - _NPG), (0, 0), (0, 0)))
    base = base.transpose(1, 0, 2)
    zblk = jnp.zeros((_D_O, _NP, _D_H), jnp.float32)
    top = jnp.concatenate([base, zblk], axis=2)
    bot = jnp.concatenate([zblk, base], axis=2)
    return jnp.concatenate([top, bot], axis=1)


def kernel(x, edge_index, edge_attr, globalFeats, isTrain, W_rel1, b_rel1,
           W_root1, W_rel2, b_rel2, W_root2, Wg1, bg1, Wg2, bg2, Wg3, bg3,
           Wo1, bo1, Wo2, bo2):
    w2e = _prep_head_weights(Wo1)
    w1g = Wo1[_NPG * _D_O:]
    a = _build_adj(edge_index, edge_attr, 0, _B)
    return _net(a, x, globalFeats, W_rel1, b_rel1, W_root1,
                W_rel2, b_rel2, W_root2, Wg1, bg1, Wg2, bg2,
                Wg3, bg3, w2e, w1g, bo1, Wo2, bo2)


# speedup vs baseline: 1.2821x; 1.1299x over previous
import functools

import jax
import jax.numpy as jnp
from jax import lax
from jax.experimental import pallas as pl
from jax.experimental.pallas import tpu as pltpu
from jax.experimental.pallas import tpu_sc as plsc

_B = 1024
_NPG = 54
_NP = 64
_NR = 128
_EPG = 864
_D_IN = 64
_D_H = 128
_D_O = 4
_GLOB = 32

_G_CONV = 64


def _build_adj(edge_index, w, g0, gn):
    info = plsc.get_sparse_core_info()
    n_workers = info.num_cores * info.num_subcores
    gpw = gn // n_workers
    epw = gpw * _EPG
    mesh = plsc.VectorSubcoreMesh(core_axis_name="c", subcore_axis_name="s")

    @functools.partial(
        pl.kernel,
        out_type=jax.ShapeDtypeStruct((gn, _NP * _NR), jnp.float32),
        mesh=mesh,
        scratch_types=[
            pltpu.VMEM((epw,), jnp.int32),
            pltpu.VMEM((epw,), jnp.int32),
            pltpu.VMEM((epw,), jnp.float32),
            pltpu.VMEM((_NP * _NR,), jnp.float32),
            pltpu.VMEM((_NP * _NR,), jnp.float32),
            pltpu.SemaphoreType.DMA,
            pltpu.SemaphoreType.DMA,
            pltpu.SemaphoreType.DMA,
            pltpu.SemaphoreType.DMA,
        ],
        compiler_params=pltpu.CompilerParams(needs_layout_passes=False),
        cost_estimate=pl.CostEstimate(
            flops=2_000_000, bytes_accessed=45_000_000, transcendentals=0),
    )
    def build(ei_hbm, w_hbm, a_hbm, src_v, dst_v, w_v, acc0, acc1,
              sem_s, sem_d, sem_w, sem_o):
        wid = lax.axis_index("s") * info.num_cores + lax.axis_index("c")
        ebase = g0 * _EPG + wid * epw
        cs = pltpu.async_copy(ei_hbm.at[0, pl.ds(ebase, epw)], src_v, sem_s)
        cd = pltpu.async_copy(ei_hbm.at[1, pl.ds(ebase, epw)], dst_v, sem_d)
        cw = pltpu.async_copy(w_hbm.at[pl.ds(ebase, epw)], w_v, sem_w)

        def zero_all(j, c):
            for u in range(4):
                acc0[pl.ds(j * 64 + u * 16, 16)] = jnp.zeros(
                    (16,), jnp.float32)
                acc1[pl.ds(j * 64 + u * 16, 16)] = jnp.zeros(
                    (16,), jnp.float32)
            return c

        lax.fori_loop(0, _NP * _NR // 64, zero_all, 0)
        cs.wait()
        cd.wait()
        cw.wait()

        accs = (acc0, acc1)
        pending = [None, None]
        for gi in range(gpw):
            acc = accs[gi % 2]
            if pending[gi % 2] is not None:
                pending[gi % 2].wait()

            if gi >= 2:
                def zero_rows(d, c, acc=acc):
                    for u in range(4):
                        acc[pl.ds(d * _NR + u * 16, 16)] = jnp.zeros(
                            (16,), jnp.float32)
                    return c

                lax.fori_loop(0, _NPG, zero_rows, 0)

            g = wid * gpw + gi
            goff = (g0 + g) * _NPG
            e0 = gi * _EPG

            def edge48(i, c, acc=acc, e0=e0, goff=goff):
                for u in range(3):
                    o = e0 + i * 48 + u * 16
                    s = src_v[pl.ds(o, 16)]
                    d = dst_v[pl.ds(o, 16)]
                    ww = w_v[pl.ds(o, 16)]
                    idx = (d - goff) * _NR + (s - goff)
                    plsc.addupdate_scatter(acc, [idx], ww)
                return c

            lax.fori_loop(0, _EPG // 48, edge48, 0)
            pending[gi % 2] = pltpu.async_copy(acc, a_hbm.at[g], sem_o)
        for p in pending:
            if p is not None:
                p.wait()

    return build(edge_index, w)


def _net_body(a_ref, x_ref, gf_ref, w1cat_ref, br1_ref, w2cat_ref,
              br2_ref, wg1_ref, bg1_ref, wg2_ref, bg2_ref, wg3_ref,
              bg3_ref, w2e_ref, w1g_ref, bo1_ref, wo2_ref, bo2_ref, out_ref,
              cat1_s, cat2_s):
    pad_src = jnp.zeros((_NR - _NPG, _D_IN), jnp.float32)
    for r in range(_G_CONV):
        x_r = x_ref[pl.ds(r * _NPG, _NPG), :]
        xp128 = jnp.concatenate([x_r, pad_src], axis=0)
        a_r = jnp.reshape(a_ref[r], (_NP, _NR))
        cat1_s[pl.ds(r * _NP, _NP), :_D_IN] = jnp.dot(
            a_r, xp128, preferred_element_type=jnp.float32)
        cat1_s[pl.ds(r * _NP, _NP), _D_IN:] = xp128[:_NP]
    h1 = jnp.maximum(
        jnp.dot(cat1_s[...], w1cat_ref[...],
                preferred_element_type=jnp.float32) + br1_ref[...],
        0.0)
    cat2_s[:, _D_H:] = h1
    pad_h = jnp.zeros((_NR - _NP, _D_H), jnp.float32)
    for r in range(_G_CONV):
        a_r = jnp.reshape(a_ref[r], (_NP, _NR))
        h1p = jnp.concatenate(
            [cat2_s[pl.ds(r * _NP, _NP), _D_H:], pad_h], axis=0)
        cat2_s[pl.ds(r * _NP, _NP), :_D_H] = jnp.dot(
            a_r, h1p, preferred_element_type=jnp.float32)
    h2 = jnp.maximum(
        jnp.dot(cat2_s[...], w2cat_ref[...],
                preferred_element_type=jnp.float32) + br2_ref[...],
        0.0)
    row_ok = (lax.broadcasted_iota(jnp.int32, (_G_CONV * _NP, _D_O), 0)
              % _NP) < _NPG
    h2 = jnp.where(row_ok, h2, 0.0)

    h2t = jnp.transpose(h2, (1, 0))
    m = jnp.reshape(h2t, (_D_O, _G_CONV // 2, 2 * _NP))
    z2 = jnp.zeros((_G_CONV // 2, 2 * _D_H), jnp.float32)
    for c in range(_D_O):
        z2 = z2 + jnp.dot(m[c], w2e_ref[c],
                          preferred_element_type=jnp.float32)
    ze = jnp.reshape(z2, (_G_CONV, _D_H))

    gv = gf_ref[...]
    gv = jnp.maximum(jnp.dot(gv, wg1_ref[...],
                             preferred_element_type=jnp.float32)
                     + bg1_ref[...], 0.0)
    gv = jnp.maximum(jnp.dot(gv, wg2_ref[...],
                             preferred_element_type=jnp.float32)
                     + bg2_ref[...], 0.0)
    gv = jnp.maximum(jnp.dot(gv, wg3_ref[...],
                             preferred_element_type=jnp.float32)
                     + bg3_ref[...], 0.0)

    z = jnp.maximum(
        ze + jnp.dot(gv, w1g_ref[...], preferred_element_type=jnp.float32)
        + bo1_ref[...], 0.0)
    z = jnp.dot(z, wo2_ref[...], preferred_element_type=jnp.float32) \
        + bo2_ref[...]
    out_ref[...] = jax.nn.sigmoid(z)


def _net(a, x, gf, w_rel1, b_rel1, w_root1, w_rel2, b_rel2, w_root2,
         wg1, bg1, wg2, bg2, wg3, bg3, w2e, w1g, bo1, wo2, bo2,
         interpret=False):
    gn = a.shape[0]
    grid = (gn // _G_CONV,)
    row = lambda shape: pl.BlockSpec(shape, lambda i: (i, 0))
    full2 = lambda shape: pl.BlockSpec(shape, lambda i: (0, 0))
    w1cat = jnp.concatenate([w_rel1, w_root1], axis=0)
    w2cat = jnp.concatenate([w_rel2, w_root2], axis=0)
    return pl.pallas_call(
        _net_body,
        grid=grid,
        in_specs=[
            row((_G_CONV, _NP * _NR)),
            row((_G_CONV * _NPG, _D_IN)),
            row((_G_CONV, _GLOB)),
            full2((2 * _D_IN, _D_H)),
            full2((1, _D_H)),
            full2((2 * _D_H, _D_O)),
            full2((1, _D_O)),
            full2((_GLOB, 8)),
            full2((1, 8)),
            full2((8, 8)),
            full2((1, 8)),
            full2((8, _GLOB)),
            full2((1, _GLOB)),
            pl.BlockSpec((_D_O, _NR, 2 * _D_H), lambda i: (0, 0, 0)),
            full2((_GLOB, _D_H)),
            full2((1, _D_H)),
            full2((_D_H, 1)),
            full2((1, 1)),
        ],
        out_specs=row((_G_CONV, 1)),
        out_shape=jax.ShapeDtypeStruct((gn, 1), jnp.float32),
        scratch_shapes=[
            pltpu.VMEM((_G_CONV * _NP, 2 * _D_IN), jnp.float32),
            pltpu.VMEM((_G_CONV * _NP, 2 * _D_H), jnp.float32),
        ],
        interpret=interpret,
    )(a, x, gf, w1cat, b_rel1.reshape(1, _D_H),
      w2cat, b_rel2.reshape(1, _D_O),
      wg1, bg1.reshape(1, 8), wg2, bg2.reshape(1, 8), wg3,
      bg3.reshape(1, _GLOB), w2e, w1g, bo1.reshape(1, _D_H), wo2,
      bo2.reshape(1, 1))


def _prep_head_weights(Wo1):
    w1r = Wo1[:_NPG * _D_O].reshape(_NPG, _D_O, _D_H)
    base = jnp.pad(w1r, ((0, _NP - _NPG), (0, 0), (0, 0)))
    base = base.transpose(1, 0, 2)
    zblk = jnp.zeros((_D_O, _NP, _D_H), jnp.float32)
    top = jnp.concatenate([base, zblk], axis=2)
    bot = jnp.concatenate([zblk, base], axis=2)
    return jnp.concatenate([top, bot], axis=1)


def kernel(x, edge_index, edge_attr, globalFeats, isTrain, W_rel1, b_rel1,
           W_root1, W_rel2, b_rel2, W_root2, Wg1, bg1, Wg2, bg2, Wg3, bg3,
           Wo1, bo1, Wo2, bo2):
    w2e = _prep_head_weights(Wo1)
    w1g = Wo1[_NPG * _D_O:]
    a = _build_adj(edge_index, edge_attr, 0, _B)
    return _net(a, x, globalFeats, W_rel1, b_rel1, W_root1,
                W_rel2, b_rel2, W_root2, Wg1, bg1, Wg2, bg2,
                Wg3, bg3, w2e, w1g, bo1, Wo2, bo2)


# speedup vs baseline: 1.3501x; 1.0530x over previous
import functools

import jax
import jax.numpy as jnp
from jax import lax
from jax.experimental import pallas as pl
from jax.experimental.pallas import tpu as pltpu
from jax.experimental.pallas import tpu_sc as plsc

_B = 1024
_NPG = 54
_NP = 64
_NR = 128
_EPG = 864
_D_IN = 64
_D_H = 128
_D_O = 4
_GLOB = 32

_G_CONV = 128


def _build_adj(edge_index, w, g0, gn):
    info = plsc.get_sparse_core_info()
    n_workers = info.num_cores * info.num_subcores
    gpw = gn // n_workers
    epw = gpw * _EPG
    mesh = plsc.VectorSubcoreMesh(core_axis_name="c", subcore_axis_name="s")

    @functools.partial(
        pl.kernel,
        out_type=jax.ShapeDtypeStruct((gn, _NP * _NR), jnp.float32),
        mesh=mesh,
        scratch_types=[
            pltpu.VMEM((epw,), jnp.int32),
            pltpu.VMEM((epw,), jnp.int32),
            pltpu.VMEM((epw,), jnp.float32),
            pltpu.VMEM((_NP * _NR,), jnp.float32),
            pltpu.VMEM((_NP * _NR,), jnp.float32),
            pltpu.SemaphoreType.DMA,
            pltpu.SemaphoreType.DMA,
            pltpu.SemaphoreType.DMA,
            pltpu.SemaphoreType.DMA,
        ],
        compiler_params=pltpu.CompilerParams(needs_layout_passes=False),
        cost_estimate=pl.CostEstimate(
            flops=2_000_000, bytes_accessed=45_000_000, transcendentals=0),
    )
    def build(ei_hbm, w_hbm, a_hbm, src_v, dst_v, w_v, acc0, acc1,
              sem_s, sem_d, sem_w, sem_o):
        wid = lax.axis_index("s") * info.num_cores + lax.axis_index("c")
        ebase = g0 * _EPG + wid * epw
        cs = pltpu.async_copy(ei_hbm.at[0, pl.ds(ebase, epw)], src_v, sem_s)
        cd = pltpu.async_copy(ei_hbm.at[1, pl.ds(ebase, epw)], dst_v, sem_d)
        cw = pltpu.async_copy(w_hbm.at[pl.ds(ebase, epw)], w_v, sem_w)

        def zero_all(j, c):
            for u in range(4):
                acc0[pl.ds(j * 64 + u * 16, 16)] = jnp.zeros(
                    (16,), jnp.float32)
                acc1[pl.ds(j * 64 + u * 16, 16)] = jnp.zeros(
                    (16,), jnp.float32)
            return c

        lax.fori_loop(0, _NP * _NR // 64, zero_all, 0)
        cs.wait()
        cd.wait()
        cw.wait()

        accs = (acc0, acc1)
        pending = [None, None]
        for gi in range(gpw):
            acc = accs[gi % 2]
            if pending[gi % 2] is not None:
                pending[gi % 2].wait()

            if gi >= 2:
                def zero_rows(d, c, acc=acc):
                    for u in range(4):
                        acc[pl.ds(d * _NR + u * 16, 16)] = jnp.zeros(
                            (16,), jnp.float32)
                    return c

                lax.fori_loop(0, _NPG, zero_rows, 0)

            g = wid * gpw + gi
            goff = (g0 + g) * _NPG
            e0 = gi * _EPG

            def edge48(i, c, acc=acc, e0=e0, goff=goff):
                for u in range(3):
                    o = e0 + i * 48 + u * 16
                    s = src_v[pl.ds(o, 16)]
                    d = dst_v[pl.ds(o, 16)]
                    ww = w_v[pl.ds(o, 16)]
                    idx = (d - goff) * _NR + (s - goff)
                    plsc.addupdate_scatter(acc, [idx], ww)
                return c

            lax.fori_loop(0, _EPG // 48, edge48, 0)
            pending[gi % 2] = pltpu.async_copy(acc, a_hbm.at[g], sem_o)
        for p in pending:
            if p is not None:
                p.wait()

    return build(edge_index, w)


def _net_body(a_ref, x_ref, gf_ref, w1cat_ref, br1_ref, w2cat_ref,
              br2_ref, wg1_ref, bg1_ref, wg2_ref, bg2_ref, wg3_ref,
              bg3_ref, w2e_ref, w1g_ref, bo1_ref, wo2_ref, bo2_ref, out_ref,
              cat1_s, cat2_s):
    pad_src = jnp.zeros((_NR - _NPG, _D_IN), jnp.float32)
    for r in range(_G_CONV):
        x_r = x_ref[pl.ds(r * _NPG, _NPG), :]
        xp128 = jnp.concatenate([x_r, pad_src], axis=0)
        a_r = jnp.reshape(a_ref[r], (_NP, _NR))
        cat1_s[pl.ds(r * _NP, _NP), :_D_IN] = jnp.dot(
            a_r, xp128, preferred_element_type=jnp.float32)
        cat1_s[pl.ds(r * _NP, _NP), _D_IN:] = xp128[:_NP]
    h1 = jnp.maximum(
        jnp.dot(cat1_s[...], w1cat_ref[...],
                preferred_element_type=jnp.float32) + br1_ref[...],
        0.0)
    cat2_s[:, _D_H:] = h1
    pad_h = jnp.zeros((_NR - _NP, _D_H), jnp.float32)
    for r in range(_G_CONV):
        a_r = jnp.reshape(a_ref[r], (_NP, _NR))
        h1p = jnp.concatenate(
            [cat2_s[pl.ds(r * _NP, _NP), _D_H:], pad_h], axis=0)
        cat2_s[pl.ds(r * _NP, _NP), :_D_H] = jnp.dot(
            a_r, h1p, preferred_element_type=jnp.float32)
    h2 = jnp.maximum(
        jnp.dot(cat2_s[...], w2cat_ref[...],
                preferred_element_type=jnp.float32) + br2_ref[...],
        0.0)
    row_ok = (lax.broadcasted_iota(jnp.int32, (_G_CONV * _NP, _D_O), 0)
              % _NP) < _NPG
    h2 = jnp.where(row_ok, h2, 0.0)

    h2t = jnp.transpose(h2, (1, 0))
    m = jnp.reshape(h2t, (_D_O, _G_CONV // 2, 2 * _NP))
    z2 = jnp.zeros((_G_CONV // 2, 2 * _D_H), jnp.float32)
    for c in range(_D_O):
        z2 = z2 + jnp.dot(m[c], w2e_ref[c],
                          preferred_element_type=jnp.float32)
    ze = jnp.reshape(z2, (_G_CONV, _D_H))

    gv = gf_ref[...]
    gv = jnp.maximum(jnp.dot(gv, wg1_ref[...],
                             preferred_element_type=jnp.float32)
                     + bg1_ref[...], 0.0)
    gv = jnp.maximum(jnp.dot(gv, wg2_ref[...],
                             preferred_element_type=jnp.float32)
                     + bg2_ref[...], 0.0)
    gv = jnp.maximum(jnp.dot(gv, wg3_ref[...],
                             preferred_element_type=jnp.float32)
                     + bg3_ref[...], 0.0)

    z = jnp.maximum(
        ze + jnp.dot(gv, w1g_ref[...], preferred_element_type=jnp.float32)
        + bo1_ref[...], 0.0)
    z = jnp.dot(z, wo2_ref[...], preferred_element_type=jnp.float32) \
        + bo2_ref[...]
    out_ref[...] = jax.nn.sigmoid(z)


def _net(a, x, gf, w_rel1, b_rel1, w_root1, w_rel2, b_rel2, w_root2,
         wg1, bg1, wg2, bg2, wg3, bg3, w2e, w1g, bo1, wo2, bo2,
         interpret=False):
    gn = a.shape[0]
    grid = (gn // _G_CONV,)
    row = lambda shape: pl.BlockSpec(shape, lambda i: (i, 0))
    full2 = lambda shape: pl.BlockSpec(shape, lambda i: (0, 0))
    w1cat = jnp.concatenate([w_rel1, w_root1], axis=0)
    w2cat = jnp.concatenate([w_rel2, w_root2], axis=0)
    return pl.pallas_call(
        _net_body,
        grid=grid,
        in_specs=[
            row((_G_CONV, _NP * _NR)),
            row((_G_CONV * _NPG, _D_IN)),
            row((_G_CONV, _GLOB)),
            full2((2 * _D_IN, _D_H)),
            full2((1, _D_H)),
            full2((2 * _D_H, _D_O)),
            full2((1, _D_O)),
            full2((_GLOB, 8)),
            full2((1, 8)),
            full2((8, 8)),
            full2((1, 8)),
            full2((8, _GLOB)),
            full2((1, _GLOB)),
            pl.BlockSpec((_D_O, _NR, 2 * _D_H), lambda i: (0, 0, 0)),
            full2((_GLOB, _D_H)),
            full2((1, _D_H)),
            full2((_D_H, 1)),
            full2((1, 1)),
        ],
        out_specs=row((_G_CONV, 1)),
        out_shape=jax.ShapeDtypeStruct((gn, 1), jnp.float32),
        scratch_shapes=[
            pltpu.VMEM((_G_CONV * _NP, 2 * _D_IN), jnp.float32),
            pltpu.VMEM((_G_CONV * _NP, 2 * _D_H), jnp.float32),
        ],
        interpret=interpret,
    )(a, x, gf, w1cat, b_rel1.reshape(1, _D_H),
      w2cat, b_rel2.reshape(1, _D_O),
      wg1, bg1.reshape(1, 8), wg2, bg2.reshape(1, 8), wg3,
      bg3.reshape(1, _GLOB), w2e, w1g, bo1.reshape(1, _D_H), wo2,
      bo2.reshape(1, 1))


def _prep_head_weights(Wo1):
    w1r = Wo1[:_NPG * _D_O].reshape(_NPG, _D_O, _D_H)
    base = jnp.pad(w1r, ((0, _NP - _NPG), (0, 0), (0, 0)))
    base = base.transpose(1, 0, 2)
    zblk = jnp.zeros((_D_O, _NP, _D_H), jnp.float32)
    top = jnp.concatenate([base, zblk], axis=2)
    bot = jnp.concatenate([zblk, base], axis=2)
    return jnp.concatenate([top, bot], axis=1)


def kernel(x, edge_index, edge_attr, globalFeats, isTrain, W_rel1, b_rel1,
           W_root1, W_rel2, b_rel2, W_root2, Wg1, bg1, Wg2, bg2, Wg3, bg3,
           Wo1, bo1, Wo2, bo2):
    w2e = _prep_head_weights(Wo1)
    w1g = Wo1[_NPG * _D_O:]
    a = _build_adj(edge_index, edge_attr, 0, _B)
    return _net(a, x, globalFeats, W_rel1, b_rel1, W_root1,
                W_rel2, b_rel2, W_root2, Wg1, bg1, Wg2, bg2,
                Wg3, bg3, w2e, w1g, bo1, Wo2, bo2)


# speedup vs baseline: 1.3514x; 1.0010x over previous
import functools

import jax
import jax.numpy as jnp
from jax import lax
from jax.experimental import pallas as pl
from jax.experimental.pallas import tpu as pltpu
from jax.experimental.pallas import tpu_sc as plsc

_B = 1024
_NPG = 54
_NP = 64
_NR = 128
_EPG = 864
_D_IN = 64
_D_H = 128
_D_O = 4
_GLOB = 32

_G_CONV = 256


def _build_adj(edge_index, w, g0, gn):
    info = plsc.get_sparse_core_info()
    n_workers = info.num_cores * info.num_subcores
    gpw = gn // n_workers
    epw = gpw * _EPG
    mesh = plsc.VectorSubcoreMesh(core_axis_name="c", subcore_axis_name="s")

    @functools.partial(
        pl.kernel,
        out_type=jax.ShapeDtypeStruct((gn, _NP * _NR), jnp.float32),
        mesh=mesh,
        scratch_types=[
            pltpu.VMEM((epw,), jnp.int32),
            pltpu.VMEM((epw,), jnp.int32),
            pltpu.VMEM((epw,), jnp.float32),
            pltpu.VMEM((_NP * _NR,), jnp.float32),
            pltpu.VMEM((_NP * _NR,), jnp.float32),
            pltpu.SemaphoreType.DMA,
            pltpu.SemaphoreType.DMA,
            pltpu.SemaphoreType.DMA,
            pltpu.SemaphoreType.DMA,
        ],
        compiler_params=pltpu.CompilerParams(needs_layout_passes=False),
        cost_estimate=pl.CostEstimate(
            flops=2_000_000, bytes_accessed=45_000_000, transcendentals=0),
    )
    def build(ei_hbm, w_hbm, a_hbm, src_v, dst_v, w_v, acc0, acc1,
              sem_s, sem_d, sem_w, sem_o):
        wid = lax.axis_index("s") * info.num_cores + lax.axis_index("c")
        ebase = g0 * _EPG + wid * epw
        cs = pltpu.async_copy(ei_hbm.at[0, pl.ds(ebase, epw)], src_v, sem_s)
        cd = pltpu.async_copy(ei_hbm.at[1, pl.ds(ebase, epw)], dst_v, sem_d)
        cw = pltpu.async_copy(w_hbm.at[pl.ds(ebase, epw)], w_v, sem_w)

        def zero_all(j, c):
            for u in range(4):
                acc0[pl.ds(j * 64 + u * 16, 16)] = jnp.zeros(
                    (16,), jnp.float32)
                acc1[pl.ds(j * 64 + u * 16, 16)] = jnp.zeros(
                    (16,), jnp.float32)
            return c

        lax.fori_loop(0, _NP * _NR // 64, zero_all, 0)
        cs.wait()
        cd.wait()
        cw.wait()

        accs = (acc0, acc1)
        pending = [None, None]
        for gi in range(gpw):
            acc = accs[gi % 2]
            if pending[gi % 2] is not None:
                pending[gi % 2].wait()

            if gi >= 2:
                def zero_rows(d, c, acc=acc):
                    for u in range(4):
                        acc[pl.ds(d * _NR + u * 16, 16)] = jnp.zeros(
                            (16,), jnp.float32)
                    return c

                lax.fori_loop(0, _NPG, zero_rows, 0)

            g = wid * gpw + gi
            goff = (g0 + g) * _NPG
            e0 = gi * _EPG

            def edge48(i, c, acc=acc, e0=e0, goff=goff):
                for u in range(3):
                    o = e0 + i * 48 + u * 16
                    s = src_v[pl.ds(o, 16)]
                    d = dst_v[pl.ds(o, 16)]
                    ww = w_v[pl.ds(o, 16)]
                    idx = (d - goff) * _NR + (s - goff)
                    plsc.addupdate_scatter(acc, [idx], ww)
                return c

            lax.fori_loop(0, _EPG // 48, edge48, 0)
            pending[gi % 2] = pltpu.async_copy(acc, a_hbm.at[g], sem_o)
        for p in pending:
            if p is not None:
                p.wait()

    return build(edge_index, w)


def _net_body(a_ref, x_ref, gf_ref, w1cat_ref, br1_ref, w2cat_ref,
              br2_ref, wg1_ref, bg1_ref, wg2_ref, bg2_ref, wg3_ref,
              bg3_ref, w2e_ref, w1g_ref, bo1_ref, wo2_ref, bo2_ref, out_ref,
              cat1_s, cat2_s):
    pad_src = jnp.zeros((_NR - _NPG, _D_IN), jnp.float32)
    for r in range(_G_CONV):
        x_r = x_ref[pl.ds(r * _NPG, _NPG), :]
        xp128 = jnp.concatenate([x_r, pad_src], axis=0)
        a_r = jnp.reshape(a_ref[r], (_NP, _NR))
        cat1_s[pl.ds(r * _NP, _NP), :_D_IN] = jnp.dot(
            a_r, xp128, preferred_element_type=jnp.float32)
        cat1_s[pl.ds(r * _NP, _NP), _D_IN:] = xp128[:_NP]
    h1 = jnp.maximum(
        jnp.dot(cat1_s[...], w1cat_ref[...],
                preferred_element_type=jnp.float32) + br1_ref[...],
        0.0)
    cat2_s[:, _D_H:] = h1
    pad_h = jnp.zeros((_NR - _NP, _D_H), jnp.float32)
    for r in range(_G_CONV):
        a_r = jnp.reshape(a_ref[r], (_NP, _NR))
        h1p = jnp.concatenate(
            [cat2_s[pl.ds(r * _NP, _NP), _D_H:], pad_h], axis=0)
        cat2_s[pl.ds(r * _NP, _NP), :_D_H] = jnp.dot(
            a_r, h1p, preferred_element_type=jnp.float32)
    h2 = jnp.maximum(
        jnp.dot(cat2_s[...], w2cat_ref[...],
                preferred_element_type=jnp.float32) + br2_ref[...],
        0.0)
    row_ok = (lax.broadcasted_iota(jnp.int32, (_G_CONV * _NP, _D_O), 0)
              % _NP) < _NPG
    h2 = jnp.where(row_ok, h2, 0.0)

    h2t = jnp.transpose(h2, (1, 0))
    m = jnp.reshape(h2t, (_D_O, _G_CONV // 2, 2 * _NP))
    z2 = jnp.zeros((_G_CONV // 2, 2 * _D_H), jnp.float32)
    for c in range(_D_O):
        z2 = z2 + jnp.dot(m[c], w2e_ref[c],
                          preferred_element_type=jnp.float32)
    ze = jnp.reshape(z2, (_G_CONV, _D_H))

    gv = gf_ref[...]
    gv = jnp.maximum(jnp.dot(gv, wg1_ref[...],
                             preferred_element_type=jnp.float32)
                     + bg1_ref[...], 0.0)
    gv = jnp.maximum(jnp.dot(gv, wg2_ref[...],
                             preferred_element_type=jnp.float32)
                     + bg2_ref[...], 0.0)
    gv = jnp.maximum(jnp.dot(gv, wg3_ref[...],
                             preferred_element_type=jnp.float32)
                     + bg3_ref[...], 0.0)

    z = jnp.maximum(
        ze + jnp.dot(gv, w1g_ref[...], preferred_element_type=jnp.float32)
        + bo1_ref[...], 0.0)
    z = jnp.dot(z, wo2_ref[...], preferred_element_type=jnp.float32) \
        + bo2_ref[...]
    out_ref[...] = jax.nn.sigmoid(z)


def _net(a, x, gf, w_rel1, b_rel1, w_root1, w_rel2, b_rel2, w_root2,
         wg1, bg1, wg2, bg2, wg3, bg3, w2e, w1g, bo1, wo2, bo2,
         interpret=False):
    gn = a.shape[0]
    grid = (gn // _G_CONV,)
    row = lambda shape: pl.BlockSpec(shape, lambda i: (i, 0))
    full2 = lambda shape: pl.BlockSpec(shape, lambda i: (0, 0))
    w1cat = jnp.concatenate([w_rel1, w_root1], axis=0)
    w2cat = jnp.concatenate([w_rel2, w_root2], axis=0)
    return pl.pallas_call(
        _net_body,
        grid=grid,
        in_specs=[
            row((_G_CONV, _NP * _NR)),
            row((_G_CONV * _NPG, _D_IN)),
            row((_G_CONV, _GLOB)),
            full2((2 * _D_IN, _D_H)),
            full2((1, _D_H)),
            full2((2 * _D_H, _D_O)),
            full2((1, _D_O)),
            full2((_GLOB, 8)),
            full2((1, 8)),
            full2((8, 8)),
            full2((1, 8)),
            full2((8, _GLOB)),
            full2((1, _GLOB)),
            pl.BlockSpec((_D_O, _NR, 2 * _D_H), lambda i: (0, 0, 0)),
            full2((_GLOB, _D_H)),
            full2((1, _D_H)),
            full2((_D_H, 1)),
            full2((1, 1)),
        ],
        out_specs=row((_G_CONV, 1)),
        out_shape=jax.ShapeDtypeStruct((gn, 1), jnp.float32),
        scratch_shapes=[
            pltpu.VMEM((_G_CONV * _NP, 2 * _D_IN), jnp.float32),
            pltpu.VMEM((_G_CONV * _NP, 2 * _D_H), jnp.float32),
        ],
        interpret=interpret,
    )(a, x, gf, w1cat, b_rel1.reshape(1, _D_H),
      w2cat, b_rel2.reshape(1, _D_O),
      wg1, bg1.reshape(1, 8), wg2, bg2.reshape(1, 8), wg3,
      bg3.reshape(1, _GLOB), w2e, w1g, bo1.reshape(1, _D_H), wo2,
      bo2.reshape(1, 1))


def _prep_head_weights(Wo1):
    w1r = Wo1[:_NPG * _D_O].reshape(_NPG, _D_O, _D_H)
    base = jnp.pad(w1r, ((0, _NP - _NPG), (0, 0), (0, 0)))
    base = base.transpose(1, 0, 2)
    zblk = jnp.zeros((_D_O, _NP, _D_H), jnp.float32)
    top = jnp.concatenate([base, zblk], axis=2)
    bot = jnp.concatenate([zblk, base], axis=2)
    return jnp.concatenate([top, bot], axis=1)


def kernel(x, edge_index, edge_attr, globalFeats, isTrain, W_rel1, b_rel1,
           W_root1, W_rel2, b_rel2, W_root2, Wg1, bg1, Wg2, bg2, Wg3, bg3,
           Wo1, bo1, Wo2, bo2):
    w2e = _prep_head_weights(Wo1)
    w1g = Wo1[_NPG * _D_O:]
    a = _build_adj(edge_index, edge_attr, 0, _B)
    return _net(a, x, globalFeats, W_rel1, b_rel1, W_root1,
                W_rel2, b_rel2, W_root2, Wg1, bg1, Wg2, bg2,
                Wg3, bg3, w2e, w1g, bo1, Wo2, bo2)
